# Initial kernel scaffold; baseline (speedup 1.0000x reference)
import jax
import jax.numpy as jnp
from jax.experimental import pallas as pl


def kernel(x, edge_index, W1, b1, W2, b2, W3, b3, Wl, bl):
    raise NotImplementedError("write your pallas kernel here")


# speedup vs baseline: 4.8399x; 4.8399x over previous
import functools

import jax
import jax.numpy as jnp
from jax import lax
from jax.experimental import pallas as pl
from jax.experimental.pallas import tpu as pltpu
from jax.experimental.pallas import tpu_sc as plsc

N = 10000
NP = 10240
E = 320000
D = 128
NC = 2
NS = 16
NW = NC * NS
EPT = E // NW
CH = 80
RPT = NP // NS
BT = 1024
NG = NP // BT


def _sc_mesh():
    return plsc.VectorSubcoreMesh(core_axis_name="c", subcore_axis_name="s")


def _sc_degrees(src, dst):
    @functools.partial(
        pl.kernel,
        out_type=jax.ShapeDtypeStruct((NC, 2, NP), jnp.float32),
        mesh=_sc_mesh(),
        scratch_types=[
            pltpu.VMEM((CH,), jnp.int32),
            pltpu.VMEM((CH,), jnp.float32),
            pltpu.VMEM((RPT,), jnp.float32),
            pltpu.VMEM_SHARED((NP,), jnp.float32),
            pltpu.VMEM_SHARED((NP,), jnp.float32),
        ],
    )
    def deg_kernel(src_hbm, dst_hbm, out_hbm, idx_v, ones_v, zero_v, acc_s, acc_d):
        cid = lax.axis_index("c")
        sid = lax.axis_index("s")
        w = cid * NS + sid

        def fill_ones(i, _):
            ones_v[pl.ds(i * 16, 16)] = jnp.ones((16,), jnp.float32)
            return 0

        lax.fori_loop(0, CH // 16, fill_ones, 0)

        def fill_zero(i, _):
            zero_v[pl.ds(i * 16, 16)] = jnp.zeros((16,), jnp.float32)
            return 0

        lax.fori_loop(0, RPT // 16, fill_zero, 0)

        pltpu.sync_copy(zero_v, acc_s.at[pl.ds(sid * RPT, RPT)])
        pltpu.sync_copy(zero_v, acc_d.at[pl.ds(sid * RPT, RPT)])
        plsc.subcore_barrier()

        base = w * EPT

        def step(c, _):
            off = base + c * CH
            pltpu.sync_copy(src_hbm.at[pl.ds(off, CH)], idx_v)
            pltpu.sync_copy(ones_v, acc_s.at[idx_v], add=True)
            pltpu.sync_copy(dst_hbm.at[pl.ds(off, CH)], idx_v)
            pltpu.sync_copy(ones_v, acc_d.at[idx_v], add=True)
            return 0

        lax.fori_loop(0, EPT // CH, step, 0)
        plsc.subcore_barrier()

        r0 = sid * RPT
        pltpu.sync_copy(acc_s.at[pl.ds(r0, RPT)], out_hbm.at[cid, 0, pl.ds(r0, RPT)])
        pltpu.sync_copy(acc_d.at[pl.ds(r0, RPT)], out_hbm.at[cid, 1, pl.ds(r0, RPT)])

    return deg_kernel(src, dst)


def _sc_scatter(hw, src, dst):
    @functools.partial(
        pl.kernel,
        out_type=jax.ShapeDtypeStruct((NC, NP, D), jnp.float32),
        mesh=_sc_mesh(),
        scratch_types=[
            pltpu.VMEM((CH,), jnp.int32),
            pltpu.VMEM((CH,), jnp.int32),
            pltpu.VMEM((CH, D), jnp.float32),
            pltpu.VMEM_SHARED((NP, D), jnp.float32),
            pltpu.SemaphoreType.DMA,
        ],
    )
    def msg_kernel(hw_hbm, src_hbm, dst_hbm, out_hbm, sidx, didx, rows, acc, sem):
        cid = lax.axis_index("c")
        sid = lax.axis_index("s")
        w = cid * NS + sid

        def zrow(i, _):
            def zcol(j, _):
                rows[i, pl.ds(j * 16, 16)] = jnp.zeros((16,), jnp.float32)
                return 0

            return lax.fori_loop(0, D // 16, zcol, 0)

        lax.fori_loop(0, CH, zrow, 0)

        def zcopy(k, _):
            pltpu.sync_copy(rows, acc.at[pl.ds(sid * RPT + k * CH, CH)])
            return 0

        lax.fori_loop(0, RPT // CH, zcopy, 0)
        plsc.subcore_barrier()

        base = w * EPT

        def step(c, _):
            off = base + c * CH
            pltpu.sync_copy(src_hbm.at[pl.ds(off, CH)], sidx)
            pltpu.async_copy(hw_hbm.at[sidx], rows, sem).wait()
            pltpu.sync_copy(dst_hbm.at[pl.ds(off, CH)], didx)
            pltpu.sync_copy(rows, acc.at[didx], add=True)
            return 0

        lax.fori_loop(0, EPT // CH, step, 0)
        plsc.subcore_barrier()

        r0 = sid * RPT
        pltpu.sync_copy(acc.at[pl.ds(r0, RPT)], out_hbm.at[cid, pl.ds(r0, RPT)])

    return msg_kernel(hw, src, dst)


def _norm(degp):
    d = degp[0] + degp[1]
    return jnp.where(d > 0, lax.rsqrt(jnp.maximum(d, 1.0)), 0.0)


def _tc_first(xp, degoutp, W1):
    def body(x_ref, dop_ref, w_ref, o_ref):
        ns = _norm(dop_ref[...])
        o_ref[...] = jnp.dot(x_ref[...] * ns, w_ref[...],
                             preferred_element_type=jnp.float32)

    return pl.pallas_call(
        body,
        grid=(NG,),
        in_specs=[
            pl.BlockSpec((BT, D), lambda i: (i, 0)),
            pl.BlockSpec((NC, BT, 1), lambda i: (0, i, 0)),
            pl.BlockSpec((D, D), lambda i: (0, 0)),
        ],
        out_specs=pl.BlockSpec((BT, D), lambda i: (i, 0)),
        out_shape=jax.ShapeDtypeStruct((NP, D), jnp.float32),
    )(xp, degoutp, W1)


def _tc_mid(aggp, deginp, degoutp, b, W):
    def body(a_ref, dip_ref, dop_ref, b_ref, w_ref, o_ref):
        a = a_ref[0] + a_ref[1]
        nd = _norm(dip_ref[...])
        h = jnp.maximum(a * nd + b_ref[...][None, :], 0.0)
        ns = _norm(dop_ref[...])
        o_ref[...] = jnp.dot(h * ns, w_ref[...],
                             preferred_element_type=jnp.float32)

    return pl.pallas_call(
        body,
        grid=(NG,),
        in_specs=[
            pl.BlockSpec((NC, BT, D), lambda i: (0, i, 0)),
            pl.BlockSpec((NC, BT, 1), lambda i: (0, i, 0)),
            pl.BlockSpec((NC, BT, 1), lambda i: (0, i, 0)),
            pl.BlockSpec((D,), lambda i: (0,)),
            pl.BlockSpec((D, D), lambda i: (0, 0)),
        ],
        out_specs=pl.BlockSpec((BT, D), lambda i: (i, 0)),
        out_shape=jax.ShapeDtypeStruct((NP, D), jnp.float32),
    )(aggp, deginp, degoutp, b, W)


def _tc_final(aggp, deginp, b3, Wl, bl):
    def body(a_ref, dip_ref, b_ref, wl_ref, bl_ref, o_ref):
        i = pl.program_id(0)
        a = a_ref[0] + a_ref[1]
        nd = _norm(dip_ref[...])
        h = jnp.maximum(a * nd + b_ref[...][None, :], 0.0)
        rid = lax.broadcasted_iota(jnp.int32, (BT, 1), 0) + i * BT
        h = jnp.where(rid < N, h, 0.0)
        s = jnp.sum(h, axis=0, keepdims=True)
        p = jnp.dot(s, wl_ref[...], preferred_element_type=jnp.float32)

        @pl.when(i == 0)
        def _init():
            o_ref[...] = jnp.zeros_like(o_ref)

        o_ref[...] += p

        @pl.when(i == NG - 1)
        def _fin():
            o_ref[...] = o_ref[...] / float(N) + bl_ref[...][None, :]

    return pl.pallas_call(
        body,
        grid=(NG,),
        in_specs=[
            pl.BlockSpec((NC, BT, D), lambda i: (0, i, 0)),
            pl.BlockSpec((NC, BT, 1), lambda i: (0, i, 0)),
            pl.BlockSpec((D,), lambda i: (0,)),
            pl.BlockSpec((D, 1), lambda i: (0, 0)),
            pl.BlockSpec((1,), lambda i: (0,)),
        ],
        out_specs=pl.BlockSpec((1, 1), lambda i: (0, 0)),
        out_shape=jax.ShapeDtypeStruct((1, 1), jnp.float32),
    )(aggp, deginp, b3, Wl, bl)


def kernel(x, edge_index, W1, b1, W2, b2, W3, b3, Wl, bl):
    src = edge_index[0]
    dst = edge_index[1]
    xp = jnp.pad(x, ((0, NP - N), (0, 0)))

    degp = _sc_degrees(src, dst)
    degsrc = degp[:, 0, :].reshape(NC, NP, 1)
    degdst = degp[:, 1, :].reshape(NC, NP, 1)

    hw1 = _tc_first(xp, degsrc, W1)
    agg1 = _sc_scatter(hw1, src, dst)
    hw2 = _tc_mid(agg1, degdst, degsrc, b1, W2)
    agg2 = _sc_scatter(hw2, src, dst)
    hw3 = _tc_mid(agg2, degdst, degsrc, b2, W3)
    agg3 = _sc_scatter(hw3, src, dst)
    return _tc_final(agg3, degdst, b3, Wl, bl)
